# baseline (speedup 1.0000x reference)
import functools

import jax
import jax.numpy as jnp
from jax.experimental import pallas as pl
from jax.experimental.pallas import tpu as pltpu

INPUT_DIM = 768
D_MODEL = 768
K_DIM = 16
K = 8
R = 2
L = 2
D_U = 64
D_A = 32
D_M = 64
GAMMA = 1.0
TEMP = 1.0
P = 16
N_TOK = 4096

TILE = 2048


def _fused_body(x_ref, W_in_ref, b_in_ref, W_k_ref, anchors_t_ref,
                proxies_t_ref, W_e1_ref, W_e2_ref, b_e_ref, A_val_ref,
                W_rm_ref, W_am_ref, U_m_ref, W_mk_ref, W_y_ref, W_my_ref,
                tok_ref, y_ref):
    f32 = jnp.float32
    bf16 = jnp.bfloat16
    dot = functools.partial(jnp.dot, preferred_element_type=f32)

    x = x_ref[...]
    h = dot(x.astype(bf16), W_in_ref[...]) + b_in_ref[...]
    m = jnp.zeros((x.shape[0], D_M), f32)

    T = x.shape[0]
    iota_k = jax.lax.broadcasted_iota(jnp.int32, (K, T), 0)
    big = jnp.int32(K + 1)

    def dott(a, b):
        return jax.lax.dot_general(a, b, (((1,), (1,)), ((), ())),
                                   preferred_element_type=f32)

    def dotl(a, b):
        return jax.lax.dot_general(a, b, (((0,), (0,)), ((), ())),
                                   preferred_element_type=f32)

    row_i = jax.lax.broadcasted_iota(jnp.int32, (K, K * D_U), 0)
    col_i = jax.lax.broadcasted_iota(jnp.int32, (K, K * D_U), 1)
    expand = (row_i == col_i // D_U).astype(f32)

    for _ in range(L):
        h_bf = h.astype(bf16)
        keys = dot(h_bf, W_k_ref[...]) + dot(m, W_mk_ref[...])
        al_t = dott(anchors_t_ref[...], keys)
        pm_t = dott(proxies_t_ref[...], keys)
        proxy_t = jnp.max(pm_t.reshape(K, P, T), axis=1)
        logits_t = (al_t + GAMMA * proxy_t) / TEMP

        v1 = jnp.max(logits_t, axis=0, keepdims=True)
        i1 = jnp.min(jnp.where(logits_t == v1, iota_k, big),
                     axis=0, keepdims=True)
        one1 = (iota_k == i1)
        masked = jnp.where(one1, -jnp.inf, logits_t)
        v2 = jnp.max(masked, axis=0, keepdims=True)
        i2 = jnp.min(jnp.where(masked == v2, iota_k, big),
                     axis=0, keepdims=True)
        one2 = (iota_k == i2)
        e2 = jnp.exp(v2 - v1)
        g1 = 1.0 / (1.0 + e2)
        g2 = e2 * g1
        gates_t = (jnp.where(one1, g1, 0.0)
                   + jnp.where(one2, g2, 0.0))

        u = dot(h_bf, W_e1_ref[...])
        ug = jax.nn.gelu(u)
        scale = dotl(gates_t, expand)
        ugs = (ug * scale).astype(bf16)
        routed = dot(ugs, W_e2_ref[...]) + dotl(gates_t, b_e_ref[...])

        a = dotl(gates_t, A_val_ref[...])
        m = jnp.tanh(dot(m, U_m_ref[...])
                     + dot(routed.astype(bf16), W_rm_ref[...])
                     + dot(a, W_am_ref[...]))
        h = h + routed

    tok_ref[...] = h
    y_ref[...] = jnp.tanh(dot(h.astype(bf16), W_y_ref[...])
                          + dot(m, W_my_ref[...]))


def kernel(x, W_in, b_in, W_k, anchors, proxies, W_e1, W_e2, b_e, A_val,
           W_rm, W_am, U_m, W_mk, W_y, W_my):
    n = x.shape[0]
    anchors_t = anchors
    proxies_t = proxies.reshape(K * P, K_DIM)
    W_in = W_in.astype(jnp.bfloat16)
    W_e1_flat = W_e1.transpose(1, 0, 2).reshape(D_MODEL, K * D_U).astype(
        jnp.bfloat16)
    W_e2_flat = W_e2.reshape(K * D_U, D_MODEL).astype(jnp.bfloat16)
    W_y = W_y.astype(jnp.bfloat16)
    W_k = W_k.astype(jnp.bfloat16)
    W_rm = W_rm.astype(jnp.bfloat16)
    b_in2 = b_in.reshape(1, D_MODEL)

    grid = (n // TILE,)
    tok_spec = pl.BlockSpec((TILE, D_MODEL), lambda i: (i, 0))

    def full(shape):
        nd = len(shape)
        return pl.BlockSpec(shape, lambda i, _nd=nd: (0,) * _nd)

    out_shape = (jax.ShapeDtypeStruct((n, D_MODEL), x.dtype),
                 jax.ShapeDtypeStruct((n, D_MODEL), x.dtype))

    tokens, y_star = pl.pallas_call(
        _fused_body,
        grid=grid,
        in_specs=[
            tok_spec,
            full((D_MODEL, D_MODEL)),
            full((1, D_MODEL)),
            full((D_MODEL, K_DIM)),
            full((K, K_DIM)),
            full((K * P, K_DIM)),
            full((D_MODEL, K * D_U)),
            full((K * D_U, D_MODEL)),
            full((K, D_MODEL)),
            full((K, D_A)),
            full((D_MODEL, D_M)),
            full((D_A, D_M)),
            full((D_M, D_M)),
            full((D_M, K_DIM)),
            full((D_MODEL, D_MODEL)),
            full((D_M, D_MODEL)),
        ],
        out_specs=(tok_spec, tok_spec),
        out_shape=out_shape,
    )(x, W_in, b_in2, W_k, anchors_t, proxies_t, W_e1_flat, W_e2_flat,
      b_e, A_val, W_rm, W_am, U_m, W_mk, W_y, W_my)
    return tokens, y_star


# speedup vs baseline: 1.0324x; 1.0324x over previous
import functools

import jax
import jax.numpy as jnp
from jax.experimental import pallas as pl
from jax.experimental.pallas import tpu as pltpu

INPUT_DIM = 768
D_MODEL = 768
K_DIM = 16
K = 8
R = 2
L = 2
D_U = 64
D_A = 32
D_M = 64
GAMMA = 1.0
TEMP = 1.0
P = 16
N_TOK = 4096

TILE = 1024


def _fused_body(x_ref, W_in_ref, b_in_ref, W_k_ref, anchors_ref,
                proxies_ref, W_e1_ref, W_e2_ref, b_e_ref, A_val_ref,
                W_rm_ref, W_am_ref, U_m_ref, W_mk_ref, W_y_ref, W_my_ref,
                tok_ref, y_ref,
                w_in_s, w_k_s, w_e1_s, w_e2_s, w_rm_s, w_y_s):
    f32 = jnp.float32
    bf16 = jnp.bfloat16
    dot = functools.partial(jnp.dot, preferred_element_type=f32)

    @pl.when(pl.program_id(0) == 0)
    def _stage():
        w_in_s[...] = W_in_ref[...].astype(bf16)
        w_k_s[...] = W_k_ref[...].astype(bf16)
        w_e2_s[...] = W_e2_ref[...].astype(bf16)
        w_rm_s[...] = W_rm_ref[...].astype(bf16)
        w_y_s[...] = W_y_ref[...].astype(bf16)
        for k in range(K):
            w_e1_s[:, k * D_U:(k + 1) * D_U] = W_e1_ref[k].astype(bf16)

    x = x_ref[...]
    h = dot(x.astype(bf16), w_in_s[...]) + b_in_ref[...]
    m = jnp.zeros((x.shape[0], D_M), f32)

    T = x.shape[0]
    iota_k = jax.lax.broadcasted_iota(jnp.int32, (K, T), 0)
    big = jnp.int32(K + 1)

    def dott(a, b):
        return jax.lax.dot_general(a, b, (((1,), (1,)), ((), ())),
                                   preferred_element_type=f32)

    def dotl(a, b):
        return jax.lax.dot_general(a, b, (((0,), (0,)), ((), ())),
                                   preferred_element_type=f32)

    row_i = jax.lax.broadcasted_iota(jnp.int32, (K, K * D_U), 0)
    col_i = jax.lax.broadcasted_iota(jnp.int32, (K, K * D_U), 1)
    expand = (row_i == col_i // D_U).astype(f32)

    for _ in range(L):
        h_bf = h.astype(bf16)
        keys = dot(h_bf, w_k_s[...]) + dot(m, W_mk_ref[...])
        al_t = dott(anchors_ref[...], keys)
        pm_t = dott(proxies_ref[...], keys)
        proxy_t = jnp.max(pm_t.reshape(K, P, T), axis=1)
        logits_t = (al_t + GAMMA * proxy_t) / TEMP

        v1 = jnp.max(logits_t, axis=0, keepdims=True)
        i1 = jnp.min(jnp.where(logits_t == v1, iota_k, big),
                     axis=0, keepdims=True)
        one1 = (iota_k == i1)
        masked = jnp.where(one1, -jnp.inf, logits_t)
        v2 = jnp.max(masked, axis=0, keepdims=True)
        i2 = jnp.min(jnp.where(masked == v2, iota_k, big),
                     axis=0, keepdims=True)
        one2 = (iota_k == i2)
        e2 = jnp.exp(v2 - v1)
        g1 = 1.0 / (1.0 + e2)
        g2 = e2 * g1
        gates_t = (jnp.where(one1, g1, 0.0)
                   + jnp.where(one2, g2, 0.0))

        u = dot(h_bf, w_e1_s[...])
        ug = jax.nn.gelu(u)
        scale = dotl(gates_t, expand)
        ugs = (ug * scale).astype(bf16)
        routed = dot(ugs, w_e2_s[...]) + dotl(gates_t, b_e_ref[...])

        a = dotl(gates_t, A_val_ref[...])
        m = jnp.tanh(dot(m, U_m_ref[...])
                     + dot(routed.astype(bf16), w_rm_s[...])
                     + dot(a, W_am_ref[...]))
        h = h + routed

    tok_ref[...] = h
    y_ref[...] = jnp.tanh(dot(h.astype(bf16), w_y_s[...])
                          + dot(m, W_my_ref[...]))


def kernel(x, W_in, b_in, W_k, anchors, proxies, W_e1, W_e2, b_e, A_val,
           W_rm, W_am, U_m, W_mk, W_y, W_my):
    n = x.shape[0]
    proxies_t = proxies.reshape(K * P, K_DIM)
    W_e2_flat = W_e2.reshape(K * D_U, D_MODEL)
    b_in2 = b_in.reshape(1, D_MODEL)

    grid = (n // TILE,)
    tok_spec = pl.BlockSpec((TILE, D_MODEL), lambda i: (i, 0))

    def full(shape):
        nd = len(shape)
        return pl.BlockSpec(shape, lambda i, _nd=nd: (0,) * _nd)

    out_shape = (jax.ShapeDtypeStruct((n, D_MODEL), x.dtype),
                 jax.ShapeDtypeStruct((n, D_MODEL), x.dtype))

    bf16 = jnp.bfloat16
    tokens, y_star = pl.pallas_call(
        _fused_body,
        grid=grid,
        in_specs=[
            tok_spec,
            full((D_MODEL, D_MODEL)),
            full((1, D_MODEL)),
            full((D_MODEL, K_DIM)),
            full((K, K_DIM)),
            full((K * P, K_DIM)),
            full((K, D_MODEL, D_U)),
            full((K * D_U, D_MODEL)),
            full((K, D_MODEL)),
            full((K, D_A)),
            full((D_MODEL, D_M)),
            full((D_A, D_M)),
            full((D_M, D_M)),
            full((D_M, K_DIM)),
            full((D_MODEL, D_MODEL)),
            full((D_M, D_MODEL)),
        ],
        out_specs=(tok_spec, tok_spec),
        out_shape=out_shape,
        scratch_shapes=[
            pltpu.VMEM((D_MODEL, D_MODEL), bf16),
            pltpu.VMEM((D_MODEL, K_DIM), bf16),
            pltpu.VMEM((D_MODEL, K * D_U), bf16),
            pltpu.VMEM((K * D_U, D_MODEL), bf16),
            pltpu.VMEM((D_MODEL, D_M), bf16),
            pltpu.VMEM((D_MODEL, D_MODEL), bf16),
        ],
    )(x, W_in, b_in2, W_k, anchors, proxies_t, W_e1, W_e2_flat,
      b_e, A_val, W_rm, W_am, U_m, W_mk, W_y, W_my)
    return tokens, y_star


# speedup vs baseline: 1.0734x; 1.0397x over previous
import functools

import jax
import jax.numpy as jnp
from jax.experimental import pallas as pl
from jax.experimental.pallas import tpu as pltpu

INPUT_DIM = 768
D_MODEL = 768
K_DIM = 16
K = 8
R = 2
L = 2
D_U = 64
D_A = 32
D_M = 64
GAMMA = 1.0
TEMP = 1.0
P = 16
N_TOK = 4096

TILE = 1024
CRW = 256
RM_OFF = 128


def _fused_body(x_ref, W_in_ref, b_in_ref, W_k_ref, anchors_ref,
                proxies_ref, W_e1_ref, W_e2_ref, b_e_ref, A_val_ref,
                W_rm_ref, W_am_ref, U_m_ref, W_mk_ref, W_y_ref, W_my_ref,
                tok_ref, y_ref,
                w_in_s, w_k_s, w_e1_s, w_e2_s, w_y_s, w_my_s,
                w_krm_s, w_e2krm_s, be_krm_s):
    f32 = jnp.float32
    bf16 = jnp.bfloat16
    dot = functools.partial(jnp.dot, preferred_element_type=f32)

    @pl.when(pl.program_id(0) == 0)
    def _stage():
        w_in_s[...] = W_in_ref[...].astype(bf16)
        w_k_s[...] = W_k_ref[...].astype(bf16)
        w_e2_s[...] = W_e2_ref[...].astype(bf16)
        w_y_s[...] = W_y_ref[...].astype(bf16)
        w_my_s[...] = W_my_ref[...].astype(bf16)
        for k in range(K):
            w_e1_s[:, k * D_U:(k + 1) * D_U] = W_e1_ref[k].astype(bf16)
        w_krm_s[...] = jnp.zeros((D_MODEL, CRW), bf16)
        w_krm_s[:, 0:K_DIM] = W_k_ref[...].astype(bf16)
        w_krm_s[:, RM_OFF:RM_OFF + D_M] = W_rm_ref[...].astype(bf16)
        w_e2krm_s[...] = dot(w_e2_s[...], w_krm_s[...]).astype(bf16)
        be_krm_s[...] = dot(b_e_ref[...].astype(bf16), w_krm_s[...])

    x = x_ref[...]
    h = dot(x.astype(bf16), w_in_s[...]) + b_in_ref[...]

    T = x.shape[0]
    iota_k = jax.lax.broadcasted_iota(jnp.int32, (K, T), 0)
    big = jnp.int32(K + 1)

    def dott(a, b):
        return jax.lax.dot_general(a, b, (((1,), (1,)), ((), ())),
                                   preferred_element_type=f32)

    def dotl(a, b):
        return jax.lax.dot_general(a, b, (((0,), (0,)), ((), ())),
                                   preferred_element_type=f32)

    row_i = jax.lax.broadcasted_iota(jnp.int32, (K, K * D_U), 0)
    col_i = jax.lax.broadcasted_iota(jnp.int32, (K, K * D_U), 1)
    expand = (row_i == col_i // D_U).astype(f32)

    def route_and_expert(h_bf, keys):
        al_t = dott(anchors_ref[...], keys)
        pm_t = dott(proxies_ref[...], keys)
        proxy_t = jnp.max(pm_t.reshape(K, P, T), axis=1)
        logits_t = (al_t + GAMMA * proxy_t) / TEMP

        v1 = jnp.max(logits_t, axis=0, keepdims=True)
        i1 = jnp.min(jnp.where(logits_t == v1, iota_k, big),
                     axis=0, keepdims=True)
        one1 = (iota_k == i1)
        masked = jnp.where(one1, -jnp.inf, logits_t)
        v2 = jnp.max(masked, axis=0, keepdims=True)
        i2 = jnp.min(jnp.where(masked == v2, iota_k, big),
                     axis=0, keepdims=True)
        one2 = (iota_k == i2)
        e2 = jnp.exp(v2 - v1)
        g1 = 1.0 / (1.0 + e2)
        g2 = e2 * g1
        gates_t = (jnp.where(one1, g1, 0.0)
                   + jnp.where(one2, g2, 0.0))

        u = dot(h_bf, w_e1_s[...])
        ug = jax.nn.gelu(u)
        scale = dotl(gates_t, expand)
        ugs = (ug * scale).astype(bf16)
        routed = dot(ugs, w_e2_s[...]) + dotl(gates_t, b_e_ref[...])
        cr = dot(ugs, w_e2krm_s[...]) + dotl(gates_t, be_krm_s[...])
        a = dotl(gates_t, A_val_ref[...])
        return routed, cr, a

    h_bf = h.astype(bf16)
    keys1 = dot(h_bf, w_k_s[...])
    routed1, cr1, a1 = route_and_expert(h_bf, keys1)
    m = jnp.tanh(cr1[:, RM_OFF:RM_OFF + D_M] + dot(a1, W_am_ref[...]))
    h = h + routed1

    h_bf = h.astype(bf16)
    keys2 = keys1 + cr1[:, 0:K_DIM] + dot(m, W_mk_ref[...])
    routed2, cr2, a2 = route_and_expert(h_bf, keys2)
    m = jnp.tanh(dot(m, U_m_ref[...]) + cr2[:, RM_OFF:RM_OFF + D_M]
                 + dot(a2, W_am_ref[...]))
    h = h + routed2

    tok_ref[...] = h
    y_ref[...] = jnp.tanh(dot(h.astype(bf16), w_y_s[...])
                          + dot(m.astype(bf16), w_my_s[...]))


def kernel(x, W_in, b_in, W_k, anchors, proxies, W_e1, W_e2, b_e, A_val,
           W_rm, W_am, U_m, W_mk, W_y, W_my):
    n = x.shape[0]
    proxies_t = proxies.reshape(K * P, K_DIM)
    W_e2_flat = W_e2.reshape(K * D_U, D_MODEL)
    b_in2 = b_in.reshape(1, D_MODEL)

    grid = (n // TILE,)
    tok_spec = pl.BlockSpec((TILE, D_MODEL), lambda i: (i, 0))

    def full(shape):
        nd = len(shape)
        return pl.BlockSpec(shape, lambda i, _nd=nd: (0,) * _nd)

    out_shape = (jax.ShapeDtypeStruct((n, D_MODEL), x.dtype),
                 jax.ShapeDtypeStruct((n, D_MODEL), x.dtype))

    bf16 = jnp.bfloat16
    tokens, y_star = pl.pallas_call(
        _fused_body,
        grid=grid,
        in_specs=[
            tok_spec,
            full((D_MODEL, D_MODEL)),
            full((1, D_MODEL)),
            full((D_MODEL, K_DIM)),
            full((K, K_DIM)),
            full((K * P, K_DIM)),
            full((K, D_MODEL, D_U)),
            full((K * D_U, D_MODEL)),
            full((K, D_MODEL)),
            full((K, D_A)),
            full((D_MODEL, D_M)),
            full((D_A, D_M)),
            full((D_M, D_M)),
            full((D_M, K_DIM)),
            full((D_MODEL, D_MODEL)),
            full((D_M, D_MODEL)),
        ],
        out_specs=(tok_spec, tok_spec),
        out_shape=out_shape,
        scratch_shapes=[
            pltpu.VMEM((D_MODEL, D_MODEL), bf16),
            pltpu.VMEM((D_MODEL, K_DIM), bf16),
            pltpu.VMEM((D_MODEL, K * D_U), bf16),
            pltpu.VMEM((K * D_U, D_MODEL), bf16),
            pltpu.VMEM((D_MODEL, D_MODEL), bf16),
            pltpu.VMEM((D_M, D_MODEL), bf16),
            pltpu.VMEM((D_MODEL, CRW), bf16),
            pltpu.VMEM((K * D_U, CRW), bf16),
            pltpu.VMEM((K, CRW), jnp.float32),
        ],
    )(x, W_in, b_in2, W_k, anchors, proxies_t, W_e1, W_e2_flat,
      b_e, A_val, W_rm, W_am, U_m, W_mk, W_y, W_my)
    return tokens, y_star


# speedup vs baseline: 1.1085x; 1.0327x over previous
import functools

import jax
import jax.numpy as jnp
from jax.experimental import pallas as pl
from jax.experimental.pallas import tpu as pltpu

INPUT_DIM = 768
D_MODEL = 768
K_DIM = 16
K = 8
R = 2
L = 2
D_U = 64
D_A = 32
D_M = 64
GAMMA = 1.0
TEMP = 1.0
P = 16
N_TOK = 4096

TILE = 1024
CRW = 256
RM_OFF = 128


def _fused_body(x_ref, W_in_ref, b_in_ref, W_k_ref, anchors_ref,
                proxies_ref, W_e1_ref, W_e2_ref, b_e_ref, A_val_ref,
                W_rm_ref, W_am_ref, U_m_ref, W_mk_ref, W_y_ref, W_my_ref,
                tok_ref, y_ref,
                w_in_s, w_k_s, w_e1_s, w_e2_s, w_y_s, w_my_s,
                w_krm_s, w_e2krm_s, be_krm_s, w_be_s, w_mkum_s):
    f32 = jnp.float32
    bf16 = jnp.bfloat16
    dot = functools.partial(jnp.dot, preferred_element_type=f32)

    @pl.when(pl.program_id(0) == 0)
    def _stage():
        w_in_s[...] = W_in_ref[...].astype(bf16)
        w_k_s[...] = W_k_ref[...].astype(bf16)
        w_e2_s[...] = W_e2_ref[...].astype(bf16)
        w_y_s[...] = W_y_ref[...].astype(bf16)
        w_my_s[...] = W_my_ref[...].astype(bf16)
        for k in range(K):
            w_e1_s[:, k * D_U:(k + 1) * D_U] = W_e1_ref[k].astype(bf16)
        w_be_s[...] = b_e_ref[...].astype(bf16)
        w_krm_s[...] = jnp.zeros((D_MODEL, CRW), bf16)
        w_krm_s[:, 0:K_DIM] = W_k_ref[...].astype(bf16)
        w_krm_s[:, RM_OFF:RM_OFF + D_M] = W_rm_ref[...].astype(bf16)
        w_e2krm_s[...] = dot(w_e2_s[...], w_krm_s[...]).astype(bf16)
        avam = dot(A_val_ref[...].astype(bf16), W_am_ref[...].astype(bf16))
        bek = dot(w_be_s[...], w_krm_s[...])
        bek = bek + jnp.pad(avam, ((0, 0), (RM_OFF, CRW - RM_OFF - D_M)))
        be_krm_s[...] = bek.astype(bf16)
        w_mkum_s[...] = jnp.zeros((D_M, CRW), bf16)
        w_mkum_s[:, 0:K_DIM] = W_mk_ref[...].astype(bf16)
        w_mkum_s[:, RM_OFF:RM_OFF + D_M] = U_m_ref[...].astype(bf16)

    x = x_ref[...]
    h = dot(x.astype(bf16), w_in_s[...]) + b_in_ref[...]

    T = x.shape[0]
    iota_k = jax.lax.broadcasted_iota(jnp.int32, (K, T), 0)
    big = jnp.int32(K + 1)

    def dott(a, b):
        return jax.lax.dot_general(a, b, (((1,), (1,)), ((), ())),
                                   preferred_element_type=f32)

    def dotl(a, b):
        return jax.lax.dot_general(a, b, (((0,), (0,)), ((), ())),
                                   preferred_element_type=f32)

    row_i = jax.lax.broadcasted_iota(jnp.int32, (K, K * D_U), 0)
    col_i = jax.lax.broadcasted_iota(jnp.int32, (K, K * D_U), 1)
    expand = (row_i == col_i // D_U).astype(bf16)

    def route_and_expert(h_bf, keys):
        al_t = dott(anchors_ref[...], keys)
        pm_t = dott(proxies_ref[...], keys)
        proxy_t = jnp.max(pm_t.reshape(K, P, T), axis=1)
        logits_t = (al_t + GAMMA * proxy_t) / TEMP

        v1 = jnp.max(logits_t, axis=0, keepdims=True)
        i1 = jnp.min(jnp.where(logits_t == v1, iota_k, big),
                     axis=0, keepdims=True)
        one1 = (iota_k == i1)
        masked = jnp.where(one1, -jnp.inf, logits_t)
        v2 = jnp.max(masked, axis=0, keepdims=True)
        i2 = jnp.min(jnp.where(masked == v2, iota_k, big),
                     axis=0, keepdims=True)
        one2 = (iota_k == i2)
        e2 = jnp.exp(v2 - v1)
        g1 = 1.0 / (1.0 + e2)
        g2 = e2 * g1
        gates_t = (jnp.where(one1, g1, 0.0)
                   + jnp.where(one2, g2, 0.0))
        gates_bf = gates_t.astype(bf16)

        u = dot(h_bf, w_e1_s[...])
        ug = jax.nn.gelu(u)
        scale = dotl(gates_bf, expand)
        ugs = (ug * scale).astype(bf16)
        routed = dot(ugs, w_e2_s[...]) + dotl(gates_bf, w_be_s[...])
        cr = dot(ugs, w_e2krm_s[...]) + dotl(gates_bf, be_krm_s[...])
        return routed, cr

    h_bf = h.astype(bf16)
    keys1 = dot(h_bf, w_k_s[...])
    routed1, cr1 = route_and_expert(h_bf, keys1)
    m = jnp.tanh(cr1[:, RM_OFF:RM_OFF + D_M])
    h = h + routed1

    h_bf = h.astype(bf16)
    mcomb = dot(m.astype(bf16), w_mkum_s[...])
    keys2 = keys1 + cr1[:, 0:K_DIM] + mcomb[:, 0:K_DIM]
    routed2, cr2 = route_and_expert(h_bf, keys2)
    m = jnp.tanh(mcomb[:, RM_OFF:RM_OFF + D_M]
                 + cr2[:, RM_OFF:RM_OFF + D_M])
    h = h + routed2

    tok_ref[...] = h
    y_ref[...] = jnp.tanh(dot(h.astype(bf16), w_y_s[...])
                          + dot(m.astype(bf16), w_my_s[...]))


def kernel(x, W_in, b_in, W_k, anchors, proxies, W_e1, W_e2, b_e, A_val,
           W_rm, W_am, U_m, W_mk, W_y, W_my):
    n = x.shape[0]
    proxies_t = proxies.reshape(K * P, K_DIM)
    W_e2_flat = W_e2.reshape(K * D_U, D_MODEL)
    b_in2 = b_in.reshape(1, D_MODEL)

    grid = (n // TILE,)
    tok_spec = pl.BlockSpec((TILE, D_MODEL), lambda i: (i, 0))

    def full(shape):
        nd = len(shape)
        return pl.BlockSpec(shape, lambda i, _nd=nd: (0,) * _nd)

    out_shape = (jax.ShapeDtypeStruct((n, D_MODEL), x.dtype),
                 jax.ShapeDtypeStruct((n, D_MODEL), x.dtype))

    bf16 = jnp.bfloat16
    tokens, y_star = pl.pallas_call(
        _fused_body,
        grid=grid,
        in_specs=[
            tok_spec,
            full((D_MODEL, D_MODEL)),
            full((1, D_MODEL)),
            full((D_MODEL, K_DIM)),
            full((K, K_DIM)),
            full((K * P, K_DIM)),
            full((K, D_MODEL, D_U)),
            full((K * D_U, D_MODEL)),
            full((K, D_MODEL)),
            full((K, D_A)),
            full((D_MODEL, D_M)),
            full((D_A, D_M)),
            full((D_M, D_M)),
            full((D_M, K_DIM)),
            full((D_MODEL, D_MODEL)),
            full((D_M, D_MODEL)),
        ],
        out_specs=(tok_spec, tok_spec),
        out_shape=out_shape,
        scratch_shapes=[
            pltpu.VMEM((D_MODEL, D_MODEL), bf16),
            pltpu.VMEM((D_MODEL, K_DIM), bf16),
            pltpu.VMEM((D_MODEL, K * D_U), bf16),
            pltpu.VMEM((K * D_U, D_MODEL), bf16),
            pltpu.VMEM((D_MODEL, D_MODEL), bf16),
            pltpu.VMEM((D_M, D_MODEL), bf16),
            pltpu.VMEM((D_MODEL, CRW), bf16),
            pltpu.VMEM((K * D_U, CRW), bf16),
            pltpu.VMEM((K, CRW), bf16),
            pltpu.VMEM((K, D_MODEL), bf16),
            pltpu.VMEM((D_M, CRW), bf16),
        ],
    )(x, W_in, b_in2, W_k, anchors, proxies_t, W_e1, W_e2_flat,
      b_e, A_val, W_rm, W_am, U_m, W_mk, W_y, W_my)
    return tokens, y_star
